# baseline (speedup 1.0000x reference)
import jax
import jax.numpy as jnp
from jax import lax
from jax.experimental import pallas as pl
from jax.experimental.pallas import tpu as pltpu
from jax.experimental.pallas import tpu_sc as plsc

N = 10000
NP = 10240
HR = NP // 128
F = 128
E = 160000
NC = 2
NS = 16
EPT = E // NS
CH = 128
NCH = 79
PH = 40
EPAD = NCH * CH
ROWS_PT = NP // NS
DUMMY = NP - 1

DEC = 100000
NW = NC * NS
DPT = DEC // NW
DPTP = 3200
DCH = DPTP // 16

_MESH = plsc.VectorSubcoreMesh(core_axis_name="c", subcore_axis_name="s")
_SC_PARAMS = pltpu.CompilerParams(needs_layout_passes=False)


def _pad_tile_idx(idx, pad_value):
    t = idx.reshape(NS, EPT)
    t = jnp.pad(t, ((0, 0), (0, EPAD - EPT)), constant_values=pad_value)
    return t.reshape(NS, NCH, CH).astype(jnp.int32)


def _hist_body(srcl, dstl, zflat, out, sidx_v, didx_v, loc_s, loc_d, mrg_v,
               hs_sh, hd_sh):
    c = lax.axis_index("c")
    s = lax.axis_index("s")

    @pl.when(s == 0)
    def _():
        pltpu.sync_copy(zflat, hs_sh)
        pltpu.sync_copy(zflat, hd_sh)

    pltpu.sync_copy(zflat, loc_s)
    pltpu.sync_copy(zflat, loc_d)
    pltpu.sync_copy(srcl.at[c, s], sidx_v)
    pltpu.sync_copy(dstl.at[c, s], didx_v)
    iota = lax.iota(jnp.int32, 16)
    for k in range(HR // 16):
        mrg_v[pl.ds(k * 16, 16)] = iota + (k * 16)
    one16 = jnp.ones((16,), jnp.float32)
    plsc.subcore_barrier()

    def srcloop(j, _):
        idx = sidx_v[j // 8, pl.ds((j % 8) * 16, 16)]
        plsc.addupdate_scatter(loc_s, [idx >> 7, idx & 127], one16)
        return _

    lax.fori_loop(0, NCH * 8, srcloop, 0)

    def dstloop(j, _):
        idx = didx_v[j // 8, pl.ds((j % 8) * 16, 16)]
        plsc.addupdate_scatter(loc_d, [idx >> 7, idx & 127], one16)
        return _

    lax.fori_loop(0, NCH * 8, dstloop, 0)
    pltpu.sync_copy(loc_s, hs_sh.at[mrg_v], add=True)
    pltpu.sync_copy(loc_d, hd_sh.at[mrg_v], add=True)
    plsc.subcore_barrier()

    @pl.when(s < HR // 8)
    def _():
        pltpu.sync_copy(hs_sh.at[pl.ds(s * 8, 8)], out.at[c, 0, pl.ds(s * 8, 8)])
        pltpu.sync_copy(hd_sh.at[pl.ds(s * 8, 8)], out.at[c, 1, pl.ds(s * 8, 8)])


_hist_call = pl.kernel(
    _hist_body,
    out_type=jax.ShapeDtypeStruct((2, 2, HR, 128), jnp.float32),
    compiler_params=_SC_PARAMS,
    mesh=_MESH,
    scratch_types=[
        pltpu.VMEM((NCH, CH), jnp.int32),
        pltpu.VMEM((NCH, CH), jnp.int32),
        pltpu.VMEM((HR, 128), jnp.float32),
        pltpu.VMEM((HR, 128), jnp.float32),
        pltpu.VMEM((HR,), jnp.int32),
        pltpu.VMEM_SHARED((HR, 128), jnp.float32),
        pltpu.VMEM_SHARED((HR, 128), jnp.float32),
    ],
)


def _prop_body(featflat, srcg, dstl, z128_hbm, out, sidx_v, didx_v, rows_a,
               rows_b, acc_sh, sem_a, sem_b, sem_sa, sem_sb):
    c = lax.axis_index("c")
    s = lax.axis_index("s")
    base = s * ROWS_PT
    pltpu.sync_copy(z128_hbm, acc_sh.at[pl.ds(base, ROWS_PT)])
    plsc.subcore_barrier()

    def gather(j, buf, sem):
        pltpu.async_copy(featflat.at[sidx_v.at[j]], buf, sem)

    def wait(j, buf, sem):
        pltpu.make_async_copy(featflat.at[sidx_v.at[j]], buf, sem).wait()

    def scatter(j, buf):
        pltpu.sync_copy(buf, acc_sh.at[didx_v.at[j]], add=True)

    def scatter_async(j, buf, sem):
        pltpu.async_copy(buf, acc_sh.at[didx_v.at[j]], sem, add=True)

    def scatter_wait(buf, sem):
        pltpu.make_async_copy(buf, acc_sh.at[didx_v.at[0]], sem).wait()

    for ph_base, nk in ((0, PH), (PH, NCH - PH)):
        pltpu.sync_copy(srcg.at[c, s, pl.ds(ph_base, nk)], sidx_v.at[pl.ds(0, nk)])
        pltpu.sync_copy(dstl.at[c, s, pl.ds(ph_base, nk)], didx_v.at[pl.ds(0, nk)])
        gather(0, rows_a, sem_a)

        @pl.when(1 < nk)
        def _g1():
            gather(1, rows_b, sem_b)

        def pair(i, carry):
            j0 = 2 * i
            wait(j0, rows_a, sem_a)
            scatter_async(j0, rows_a, sem_sa)

            @pl.when(j0 + 1 < nk)
            def _sc_b():
                wait(j0 + 1, rows_b, sem_b)
                scatter_async(j0 + 1, rows_b, sem_sb)

            @pl.when(j0 + 2 < nk)
            def _pf_a():
                scatter_wait(rows_a, sem_sa)
                gather(j0 + 2, rows_a, sem_a)

            @pl.when(j0 + 3 < nk)
            def _pf_b():
                scatter_wait(rows_b, sem_sb)
                gather(j0 + 3, rows_b, sem_b)

            return carry

        lax.fori_loop(0, (nk + 1) // 2, pair, 0)
        scatter_wait(rows_a, sem_sa)

        @pl.when(1 < nk)
        def _dr_b():
            scatter_wait(rows_b, sem_sb)

    plsc.subcore_barrier()
    pltpu.sync_copy(acc_sh.at[pl.ds(base, ROWS_PT)], out.at[c, pl.ds(base, ROWS_PT)])


_prop_call = pl.kernel(
    _prop_body,
    out_type=jax.ShapeDtypeStruct((2, NP, F), jnp.float32),
    compiler_params=_SC_PARAMS,
    mesh=_MESH,
    scratch_types=[
        pltpu.VMEM((PH, CH), jnp.int32),
        pltpu.VMEM((PH, CH), jnp.int32),
        pltpu.VMEM((CH, F), jnp.float32),
        pltpu.VMEM((CH, F), jnp.float32),
        pltpu.VMEM_SHARED((NP, F), jnp.float32),
        pltpu.SemaphoreType.DMA,
        pltpu.SemaphoreType.DMA,
        pltpu.SemaphoreType.DMA,
        pltpu.SemaphoreType.DMA,
    ],
)


def _pred_body(t_hbm, es_hbm, ed_hbm, out0, out1, t_v, es_v, ed_v, s0_v, s1_v):
    c = lax.axis_index("c")
    s = lax.axis_index("s")
    w = s * NC + c
    pltpu.sync_copy(t_hbm, t_v)
    pltpu.sync_copy(es_hbm.at[w], es_v)
    pltpu.sync_copy(ed_hbm.at[w], ed_v)

    def chunk(j, _):
        e4 = es_v[j] * 4
        d4 = ed_v[j] * 4
        g0 = plsc.load_gather(t_v, [e4])
        g1 = plsc.load_gather(t_v, [e4 + 1])
        g2 = plsc.load_gather(t_v, [d4 + 2])
        g3 = plsc.load_gather(t_v, [d4 + 3])
        s0_v[pl.ds(j * 16, 16)] = g0 + g2
        s1_v[pl.ds(j * 16, 16)] = g1 + g3
        return _

    lax.fori_loop(0, DCH, chunk, 0)
    pltpu.sync_copy(s0_v, out0.at[pl.ds(w * DPTP, DPTP)])
    pltpu.sync_copy(s1_v, out1.at[pl.ds(w * DPTP, DPTP)])


_pred_call = pl.kernel(
    _pred_body,
    out_type=(
        jax.ShapeDtypeStruct((NW * DPTP,), jnp.float32),
        jax.ShapeDtypeStruct((NW * DPTP,), jnp.float32),
    ),
    compiler_params=_SC_PARAMS,
    mesh=_MESH,
    scratch_types=[
        pltpu.VMEM((NP * 4,), jnp.float32),
        pltpu.VMEM((DCH, 16), jnp.int32),
        pltpu.VMEM((DCH, 16), jnp.int32),
        pltpu.VMEM((DPTP,), jnp.float32),
        pltpu.VMEM((DPTP,), jnp.float32),
    ],
)


RB = 1024
GRID = NP // RB
HB = RB // 128


def _rs_col(hv):
    return lax.rsqrt(jnp.maximum(hv, 1.0))


def _tc_feat_body(x_ref, h_ref, o_ref):
    hv = h_ref[...]
    x = x_ref[...]
    o_ref[0] = x * _rs_col(hv[0, 0])
    o_ref[1] = x * _rs_col(hv[1, 0])


def _tc_feats(xp, hist):
    return pl.pallas_call(
        _tc_feat_body,
        grid=(GRID,),
        in_specs=[
            pl.BlockSpec((RB, F), lambda i: (i, 0)),
            pl.BlockSpec((2, 2, RB, 1), lambda i: (0, 0, i, 0)),
        ],
        out_specs=pl.BlockSpec((2, RB, F), lambda i: (0, i, 0)),
        out_shape=jax.ShapeDtypeStruct((2, NP, F), jnp.float32),
    )(xp, hist)


def _tc_mid_body(a_ref, h_ref, w0_ref, w1_ref, b_ref, o_ref):
    hv = h_ref[...]
    y0 = a_ref[0] * _rs_col(hv[0, 1])
    y1 = a_ref[1] * _rs_col(hv[1, 1])
    h = jnp.dot(y0, w0_ref[...], preferred_element_type=jnp.float32)
    h += jnp.dot(y1, w1_ref[...], preferred_element_type=jnp.float32)
    h = jnp.maximum(h + b_ref[...], 0.0)
    o_ref[0] = h * _rs_col(hv[0, 0])
    o_ref[1] = h * _rs_col(hv[1, 0])


def _tc_mid(acc, hist, W1_0, W1_1, b1r):
    return pl.pallas_call(
        _tc_mid_body,
        grid=(GRID,),
        in_specs=[
            pl.BlockSpec((2, RB, F), lambda i: (0, i, 0)),
            pl.BlockSpec((2, 2, RB, 1), lambda i: (0, 0, i, 0)),
            pl.BlockSpec((F, F), lambda i: (0, 0)),
            pl.BlockSpec((F, F), lambda i: (0, 0)),
            pl.BlockSpec((1, F), lambda i: (0, 0)),
        ],
        out_specs=pl.BlockSpec((2, RB, F), lambda i: (0, i, 0)),
        out_shape=jax.ShapeDtypeStruct((2, NP, F), jnp.float32),
    )(acc, hist, W1_0, W1_1, b1r)


def _tc_final_body(a_ref, h_ref, w0_ref, w1_ref, b_ref, wp_ref, bp_ref, o_ref):
    hv = h_ref[...]
    y0 = a_ref[0] * _rs_col(hv[0, 1])
    y1 = a_ref[1] * _rs_col(hv[1, 1])
    h2 = jnp.dot(y0, w0_ref[...], preferred_element_type=jnp.float32)
    h2 += jnp.dot(y1, w1_ref[...], preferred_element_type=jnp.float32)
    h2 += b_ref[...]
    o_ref[...] = jnp.dot(h2, wp_ref[...], preferred_element_type=jnp.float32) + bp_ref[...]


def _tc_final(acc, hist, W2_0, W2_1, b2r, Wp4, bp4):
    return pl.pallas_call(
        _tc_final_body,
        grid=(GRID,),
        in_specs=[
            pl.BlockSpec((2, RB, F), lambda i: (0, i, 0)),
            pl.BlockSpec((2, 2, RB, 1), lambda i: (0, 0, i, 0)),
            pl.BlockSpec((F, F), lambda i: (0, 0)),
            pl.BlockSpec((F, F), lambda i: (0, 0)),
            pl.BlockSpec((1, F), lambda i: (0, 0)),
            pl.BlockSpec((F, 4), lambda i: (0, 0)),
            pl.BlockSpec((1, 4), lambda i: (0, 0)),
        ],
        out_specs=pl.BlockSpec((RB, 4), lambda i: (i, 0)),
        out_shape=jax.ShapeDtypeStruct((NP, 4), jnp.float32),
    )(acc, hist, W2_0, W2_1, b2r, Wp4, bp4)


def kernel(x, edge_index_rel0, edge_index_rel1, dec_edge_index,
           W1_0, b1_0, W1_1, b1_1, W2_0, b2_0, W2_1, b2_1, Wp, bp):
    xp = jnp.pad(x, ((0, NP - N), (0, 0)))

    def pack_rel(ei, r):
        src, dst = ei[0], ei[1]
        srcl = _pad_tile_idx(src, DUMMY)
        srcg = srcl + r * NP
        dstl = _pad_tile_idx(dst, DUMMY)
        return srcl, srcg, dstl

    s0l, s0g, d0l = pack_rel(edge_index_rel0, 0)
    s1l, s1g, d1l = pack_rel(edge_index_rel1, 1)
    srcl = jnp.stack([s0l, s1l])
    srcg = jnp.stack([s0g, s1g])
    dstl = jnp.stack([d0l, d1l])

    es = dec_edge_index[0].astype(jnp.int32).reshape(NW, DPT)
    ed = dec_edge_index[1].astype(jnp.int32).reshape(NW, DPT)
    es = jnp.pad(es, ((0, 0), (0, DPTP - DPT))).reshape(NW, DCH, 16)
    ed = jnp.pad(ed, ((0, 0), (0, DPTP - DPT))).reshape(NW, DCH, 16)

    zflat = jnp.zeros((HR, 128), jnp.float32)
    z128_hbm = jnp.zeros((ROWS_PT, F), jnp.float32)

    b1r = (b1_0 + b1_1).reshape(1, F)
    b2r = (b2_0 + b2_1).reshape(1, F)
    Wp4 = jnp.concatenate([Wp[:F], Wp[F:]], axis=1)
    bp4 = jnp.concatenate([bp, jnp.zeros((2,), jnp.float32)]).reshape(1, 4)

    hist = _hist_call(srcl, dstl, zflat)

    hist = hist.reshape(2, 2, NP, 1)

    feats = _tc_feats(xp, hist)
    acc1 = _prop_call(feats.reshape(2 * NP, F), srcg, dstl, z128_hbm)
    feats2 = _tc_mid(acc1, hist, W1_0, W1_1, b1r)

    acc2 = _prop_call(feats2.reshape(2 * NP, F), srcg, dstl, z128_hbm)
    t = _tc_final(acc2, hist, W2_0, W2_1, b2r, Wp4, bp4)

    o0, o1 = _pred_call(t.reshape(NP * 4), es, ed)
    o0 = o0.reshape(NW, DPTP)[:, :DPT].reshape(DEC)
    o1 = o1.reshape(NW, DPTP)[:, :DPT].reshape(DEC)
    return jnp.stack([o0, o1], axis=1)


# speedup vs baseline: 1.1418x; 1.1418x over previous
import jax
import jax.numpy as jnp
from jax import lax
from jax.experimental import pallas as pl
from jax.experimental.pallas import tpu as pltpu
from jax.experimental.pallas import tpu_sc as plsc

N = 10000
NP = 10240
HR = NP // 128
F = 128
E = 160000
NC = 2
NS = 16
EPT = E // NS
CH = 128
NCH = 79
PH = 40
EPAD = NCH * CH
ROWS_PT = NP // NS
DUMMY = NP - 1

DEC = 100000
NW = NC * NS
DPT = DEC // NW
DPTP = 3200
DCH = DPTP // 16

_MESH = plsc.VectorSubcoreMesh(core_axis_name="c", subcore_axis_name="s")
_SC_PARAMS = pltpu.CompilerParams(needs_layout_passes=False)


def _pad_tile_idx(idx, pad_value):
    t = idx.reshape(NS, EPT)
    t = jnp.pad(t, ((0, 0), (0, EPAD - EPT)), constant_values=pad_value)
    return t.reshape(NS, NCH, CH).astype(jnp.int32)


def _hist_body(srcl, dstl, zflat, out, sidx_v, didx_v, loc_s, loc_d, mrg_v,
               hs_sh, hd_sh):
    c = lax.axis_index("c")
    s = lax.axis_index("s")

    @pl.when(s == 0)
    def _():
        pltpu.sync_copy(zflat, hs_sh)
        pltpu.sync_copy(zflat, hd_sh)

    pltpu.sync_copy(zflat, loc_s)
    pltpu.sync_copy(zflat, loc_d)
    pltpu.sync_copy(srcl.at[c, s], sidx_v)
    pltpu.sync_copy(dstl.at[c, s], didx_v)
    iota = lax.iota(jnp.int32, 16)
    for k in range(HR // 16):
        mrg_v[pl.ds(k * 16, 16)] = iota + (k * 16)
    one16 = jnp.ones((16,), jnp.float32)
    plsc.subcore_barrier()

    def srcloop(j, _):
        idx = sidx_v[j // 8, pl.ds((j % 8) * 16, 16)]
        plsc.addupdate_scatter(loc_s, [idx >> 7, idx & 127], one16)
        return _

    lax.fori_loop(0, NCH * 8, srcloop, 0)

    def dstloop(j, _):
        idx = didx_v[j // 8, pl.ds((j % 8) * 16, 16)]
        plsc.addupdate_scatter(loc_d, [idx >> 7, idx & 127], one16)
        return _

    lax.fori_loop(0, NCH * 8, dstloop, 0)
    pltpu.sync_copy(loc_s, hs_sh.at[mrg_v], add=True)
    pltpu.sync_copy(loc_d, hd_sh.at[mrg_v], add=True)
    plsc.subcore_barrier()

    @pl.when(s < HR // 8)
    def _():
        pltpu.sync_copy(hs_sh.at[pl.ds(s * 8, 8)], out.at[c, 0, pl.ds(s * 8, 8)])
        pltpu.sync_copy(hd_sh.at[pl.ds(s * 8, 8)], out.at[c, 1, pl.ds(s * 8, 8)])


_hist_call = pl.kernel(
    _hist_body,
    out_type=jax.ShapeDtypeStruct((2, 2, HR, 128), jnp.float32),
    compiler_params=_SC_PARAMS,
    mesh=_MESH,
    scratch_types=[
        pltpu.VMEM((NCH, CH), jnp.int32),
        pltpu.VMEM((NCH, CH), jnp.int32),
        pltpu.VMEM((HR, 128), jnp.float32),
        pltpu.VMEM((HR, 128), jnp.float32),
        pltpu.VMEM((HR,), jnp.int32),
        pltpu.VMEM_SHARED((HR, 128), jnp.float32),
        pltpu.VMEM_SHARED((HR, 128), jnp.float32),
    ],
)


def _prop_body(featflat, srcg, dstl, z128_hbm, out, sidx_v, didx_v, rows_a,
               rows_b, acc_sh, sem_a, sem_b):
    c = lax.axis_index("c")
    s = lax.axis_index("s")
    base = s * ROWS_PT
    pltpu.sync_copy(z128_hbm, acc_sh.at[pl.ds(base, ROWS_PT)])
    plsc.subcore_barrier()

    def gather(j, buf, sem):
        pltpu.async_copy(featflat.at[c].at[sidx_v.at[j]], buf, sem)

    def wait(j, buf, sem):
        pltpu.make_async_copy(featflat.at[c].at[sidx_v.at[j]], buf, sem).wait()

    def scatter(j, buf):
        pltpu.sync_copy(buf, acc_sh.at[didx_v.at[j]], add=True)

    for ph_base, nk in ((0, PH), (PH, NCH - PH)):
        pltpu.sync_copy(srcg.at[c, s, pl.ds(ph_base, nk)], sidx_v.at[pl.ds(0, nk)])
        pltpu.sync_copy(dstl.at[c, s, pl.ds(ph_base, nk)], didx_v.at[pl.ds(0, nk)])
        gather(0, rows_a, sem_a)

        def pair(i, carry):
            j0 = 2 * i

            @pl.when(j0 + 1 < nk)
            def _pf_b():
                gather(j0 + 1, rows_b, sem_b)

            wait(j0, rows_a, sem_a)
            scatter(j0, rows_a)

            @pl.when(j0 + 1 < nk)
            def _do_b():
                @pl.when(j0 + 2 < nk)
                def _pf_a():
                    gather(j0 + 2, rows_a, sem_a)

                wait(j0 + 1, rows_b, sem_b)
                scatter(j0 + 1, rows_b)

            return carry

        lax.fori_loop(0, (nk + 1) // 2, pair, 0)
    plsc.subcore_barrier()
    pltpu.sync_copy(acc_sh.at[pl.ds(base, ROWS_PT)], out.at[c, pl.ds(base, ROWS_PT)])


_prop_call = pl.kernel(
    _prop_body,
    out_type=jax.ShapeDtypeStruct((2, NP, F), jnp.float32),
    compiler_params=_SC_PARAMS,
    mesh=_MESH,
    scratch_types=[
        pltpu.VMEM((PH, CH), jnp.int32),
        pltpu.VMEM((PH, CH), jnp.int32),
        pltpu.VMEM((CH, F), jnp.float32),
        pltpu.VMEM((CH, F), jnp.float32),
        pltpu.VMEM_SHARED((NP, F), jnp.float32),
        pltpu.SemaphoreType.DMA,
        pltpu.SemaphoreType.DMA,
    ],
)


def _pred_body(t_hbm, es_hbm, ed_hbm, out0, out1, t_v, es_v, ed_v, s0_v, s1_v):
    c = lax.axis_index("c")
    s = lax.axis_index("s")
    w = s * NC + c
    pltpu.sync_copy(t_hbm, t_v)
    pltpu.sync_copy(es_hbm.at[w], es_v)
    pltpu.sync_copy(ed_hbm.at[w], ed_v)

    def chunk(j, _):
        e4 = es_v[j] * 4
        d4 = ed_v[j] * 4
        g0 = plsc.load_gather(t_v, [e4])
        g1 = plsc.load_gather(t_v, [e4 + 1])
        g2 = plsc.load_gather(t_v, [d4 + 2])
        g3 = plsc.load_gather(t_v, [d4 + 3])
        s0_v[pl.ds(j * 16, 16)] = g0 + g2
        s1_v[pl.ds(j * 16, 16)] = g1 + g3
        return _

    lax.fori_loop(0, DCH, chunk, 0)
    pltpu.sync_copy(s0_v, out0.at[pl.ds(w * DPTP, DPTP)])
    pltpu.sync_copy(s1_v, out1.at[pl.ds(w * DPTP, DPTP)])


_pred_call = pl.kernel(
    _pred_body,
    out_type=(
        jax.ShapeDtypeStruct((NW * DPTP,), jnp.float32),
        jax.ShapeDtypeStruct((NW * DPTP,), jnp.float32),
    ),
    compiler_params=_SC_PARAMS,
    mesh=_MESH,
    scratch_types=[
        pltpu.VMEM((NP * 4,), jnp.float32),
        pltpu.VMEM((DCH, 16), jnp.int32),
        pltpu.VMEM((DCH, 16), jnp.int32),
        pltpu.VMEM((DPTP,), jnp.float32),
        pltpu.VMEM((DPTP,), jnp.float32),
    ],
)


RB = 1024
GRID = NP // RB
HB = RB // 128


def _rs_col(hv):
    return lax.rsqrt(jnp.maximum(hv, 1.0))


def _tc_feat_body(x_ref, h_ref, o_ref):
    hv = h_ref[...]
    x = x_ref[...]
    o_ref[0] = x * _rs_col(hv[0, 0])
    o_ref[1] = x * _rs_col(hv[1, 0])


def _tc_feats(xp, hist):
    return pl.pallas_call(
        _tc_feat_body,
        grid=(GRID,),
        in_specs=[
            pl.BlockSpec((RB, F), lambda i: (i, 0)),
            pl.BlockSpec((2, 2, RB, 1), lambda i: (0, 0, i, 0)),
        ],
        out_specs=pl.BlockSpec((2, RB, F), lambda i: (0, i, 0)),
        out_shape=jax.ShapeDtypeStruct((2, NP, F), jnp.float32),
    )(xp, hist)


def _tc_mid_body(a_ref, h_ref, w0_ref, w1_ref, b_ref, o_ref):
    hv = h_ref[...]
    y0 = a_ref[0] * _rs_col(hv[0, 1])
    y1 = a_ref[1] * _rs_col(hv[1, 1])
    h = jnp.dot(y0, w0_ref[...], preferred_element_type=jnp.float32)
    h += jnp.dot(y1, w1_ref[...], preferred_element_type=jnp.float32)
    h = jnp.maximum(h + b_ref[...], 0.0)
    o_ref[0] = h * _rs_col(hv[0, 0])
    o_ref[1] = h * _rs_col(hv[1, 0])


def _tc_mid(acc, hist, W1_0, W1_1, b1r):
    return pl.pallas_call(
        _tc_mid_body,
        grid=(GRID,),
        in_specs=[
            pl.BlockSpec((2, RB, F), lambda i: (0, i, 0)),
            pl.BlockSpec((2, 2, RB, 1), lambda i: (0, 0, i, 0)),
            pl.BlockSpec((F, F), lambda i: (0, 0)),
            pl.BlockSpec((F, F), lambda i: (0, 0)),
            pl.BlockSpec((1, F), lambda i: (0, 0)),
        ],
        out_specs=pl.BlockSpec((2, RB, F), lambda i: (0, i, 0)),
        out_shape=jax.ShapeDtypeStruct((2, NP, F), jnp.float32),
    )(acc, hist, W1_0, W1_1, b1r)


def _tc_final_body(a_ref, h_ref, w0_ref, w1_ref, b_ref, wp_ref, bp_ref, o_ref):
    hv = h_ref[...]
    y0 = a_ref[0] * _rs_col(hv[0, 1])
    y1 = a_ref[1] * _rs_col(hv[1, 1])
    h2 = jnp.dot(y0, w0_ref[...], preferred_element_type=jnp.float32)
    h2 += jnp.dot(y1, w1_ref[...], preferred_element_type=jnp.float32)
    h2 += b_ref[...]
    o_ref[...] = jnp.dot(h2, wp_ref[...], preferred_element_type=jnp.float32) + bp_ref[...]


def _tc_final(acc, hist, W2_0, W2_1, b2r, Wp4, bp4):
    return pl.pallas_call(
        _tc_final_body,
        grid=(GRID,),
        in_specs=[
            pl.BlockSpec((2, RB, F), lambda i: (0, i, 0)),
            pl.BlockSpec((2, 2, RB, 1), lambda i: (0, 0, i, 0)),
            pl.BlockSpec((F, F), lambda i: (0, 0)),
            pl.BlockSpec((F, F), lambda i: (0, 0)),
            pl.BlockSpec((1, F), lambda i: (0, 0)),
            pl.BlockSpec((F, 4), lambda i: (0, 0)),
            pl.BlockSpec((1, 4), lambda i: (0, 0)),
        ],
        out_specs=pl.BlockSpec((RB, 4), lambda i: (i, 0)),
        out_shape=jax.ShapeDtypeStruct((NP, 4), jnp.float32),
    )(acc, hist, W2_0, W2_1, b2r, Wp4, bp4)


def kernel(x, edge_index_rel0, edge_index_rel1, dec_edge_index,
           W1_0, b1_0, W1_1, b1_1, W2_0, b2_0, W2_1, b2_1, Wp, bp):
    xp = jnp.pad(x, ((0, NP - N), (0, 0)))

    def pack_rel(ei):
        return _pad_tile_idx(ei[0], DUMMY), _pad_tile_idx(ei[1], DUMMY)

    s0l, d0l = pack_rel(edge_index_rel0)
    s1l, d1l = pack_rel(edge_index_rel1)
    srcl = jnp.stack([s0l, s1l])
    dstl = jnp.stack([d0l, d1l])

    es = dec_edge_index[0].astype(jnp.int32).reshape(NW, DPT)
    ed = dec_edge_index[1].astype(jnp.int32).reshape(NW, DPT)
    es = jnp.pad(es, ((0, 0), (0, DPTP - DPT))).reshape(NW, DCH, 16)
    ed = jnp.pad(ed, ((0, 0), (0, DPTP - DPT))).reshape(NW, DCH, 16)

    zflat = jnp.zeros((HR, 128), jnp.float32)
    z128_hbm = jnp.zeros((ROWS_PT, F), jnp.float32)

    b1r = (b1_0 + b1_1).reshape(1, F)
    b2r = (b2_0 + b2_1).reshape(1, F)
    Wp4 = jnp.concatenate([Wp[:F], Wp[F:]], axis=1)
    bp4 = jnp.concatenate([bp, jnp.zeros((2,), jnp.float32)]).reshape(1, 4)

    hist = _hist_call(srcl, dstl, zflat)

    hist = hist.reshape(2, 2, NP, 1)

    feats = _tc_feats(xp, hist)
    acc1 = _prop_call(feats, srcl, dstl, z128_hbm)
    feats2 = _tc_mid(acc1, hist, W1_0, W1_1, b1r)

    acc2 = _prop_call(feats2, srcl, dstl, z128_hbm)
    t = _tc_final(acc2, hist, W2_0, W2_1, b2r, Wp4, bp4)

    o0, o1 = _pred_call(t.reshape(NP * 4), es, ed)
    o0 = o0.reshape(NW, DPTP)[:, :DPT].reshape(DEC)
    o1 = o1.reshape(NW, DPTP)[:, :DPT].reshape(DEC)
    return jnp.stack([o0, o1], axis=1)


# speedup vs baseline: 1.1443x; 1.0022x over previous
import jax
import jax.numpy as jnp
from jax import lax
from jax.experimental import pallas as pl
from jax.experimental.pallas import tpu as pltpu
from jax.experimental.pallas import tpu_sc as plsc

N = 10000
NP = 10240
HR = NP // 128
F = 128
E = 160000
NC = 2
NS = 16
EPT = E // NS
CH = 128
NCH = 79
PH = 40
EPAD = NCH * CH
ROWS_PT = NP // NS
DUMMY = NP - 1

DEC = 100000
NW = NC * NS
DPT = DEC // NW
DPTP = 3200
DCH = DPTP // 16

_MESH = plsc.VectorSubcoreMesh(core_axis_name="c", subcore_axis_name="s")
_SC_PARAMS = pltpu.CompilerParams(needs_layout_passes=False)


def _pad_tile_idx(idx, pad_value):
    t = idx.reshape(NS, EPT)
    t = jnp.pad(t, ((0, 0), (0, EPAD - EPT)), constant_values=pad_value)
    return t.reshape(NS, NCH, CH).astype(jnp.int32)


def _hist_body(srcl, dstl, zflat, out, sidx_v, didx_v, loc_s, loc_d, mrg_v,
               hs_sh, hd_sh):
    c = lax.axis_index("c")
    s = lax.axis_index("s")

    @pl.when(s == 0)
    def _():
        pltpu.sync_copy(zflat, hs_sh)
        pltpu.sync_copy(zflat, hd_sh)

    pltpu.sync_copy(zflat, loc_s)
    pltpu.sync_copy(zflat, loc_d)
    pltpu.sync_copy(srcl.at[c, s], sidx_v)
    pltpu.sync_copy(dstl.at[c, s], didx_v)
    iota = lax.iota(jnp.int32, 16)
    for k in range(HR // 16):
        mrg_v[pl.ds(k * 16, 16)] = iota + (k * 16)
    one16 = jnp.ones((16,), jnp.float32)
    plsc.subcore_barrier()

    def histloop(j, carry):
        si = sidx_v[j // 8, pl.ds((j % 8) * 16, 16)]
        plsc.addupdate_scatter(loc_s, [si >> 7, si & 127], one16)
        di = didx_v[j // 8, pl.ds((j % 8) * 16, 16)]
        plsc.addupdate_scatter(loc_d, [di >> 7, di & 127], one16)
        return carry

    lax.fori_loop(0, NCH * 8, histloop, 0)
    pltpu.sync_copy(loc_s, hs_sh.at[mrg_v], add=True)
    pltpu.sync_copy(loc_d, hd_sh.at[mrg_v], add=True)
    plsc.subcore_barrier()

    @pl.when(s < HR // 8)
    def _():
        pltpu.sync_copy(hs_sh.at[pl.ds(s * 8, 8)], out.at[c, 0, pl.ds(s * 8, 8)])
        pltpu.sync_copy(hd_sh.at[pl.ds(s * 8, 8)], out.at[c, 1, pl.ds(s * 8, 8)])


_hist_call = pl.kernel(
    _hist_body,
    out_type=jax.ShapeDtypeStruct((2, 2, HR, 128), jnp.float32),
    compiler_params=_SC_PARAMS,
    mesh=_MESH,
    scratch_types=[
        pltpu.VMEM((NCH, CH), jnp.int32),
        pltpu.VMEM((NCH, CH), jnp.int32),
        pltpu.VMEM((HR, 128), jnp.float32),
        pltpu.VMEM((HR, 128), jnp.float32),
        pltpu.VMEM((HR,), jnp.int32),
        pltpu.VMEM_SHARED((HR, 128), jnp.float32),
        pltpu.VMEM_SHARED((HR, 128), jnp.float32),
    ],
)


def _prop_body(featflat, srcg, dstl, z128_hbm, out, sidx_v, didx_v, rows_a,
               rows_b, acc_sh, sem_a, sem_b):
    c = lax.axis_index("c")
    s = lax.axis_index("s")
    base = s * ROWS_PT
    pltpu.sync_copy(z128_hbm, acc_sh.at[pl.ds(base, ROWS_PT)])
    plsc.subcore_barrier()

    def gather(j, buf, sem):
        pltpu.async_copy(featflat.at[c].at[sidx_v.at[j]], buf, sem)

    def wait(j, buf, sem):
        pltpu.make_async_copy(featflat.at[c].at[sidx_v.at[j]], buf, sem).wait()

    def scatter(j, buf):
        pltpu.sync_copy(buf, acc_sh.at[didx_v.at[j]], add=True)

    for ph_base, nk in ((0, PH), (PH, NCH - PH)):
        pltpu.sync_copy(srcg.at[c, s, pl.ds(ph_base, nk)], sidx_v.at[pl.ds(0, nk)])
        pltpu.sync_copy(dstl.at[c, s, pl.ds(ph_base, nk)], didx_v.at[pl.ds(0, nk)])
        gather(0, rows_a, sem_a)

        def pair(i, carry):
            j0 = 2 * i

            @pl.when(j0 + 1 < nk)
            def _pf_b():
                gather(j0 + 1, rows_b, sem_b)

            wait(j0, rows_a, sem_a)
            scatter(j0, rows_a)

            @pl.when(j0 + 1 < nk)
            def _do_b():
                @pl.when(j0 + 2 < nk)
                def _pf_a():
                    gather(j0 + 2, rows_a, sem_a)

                wait(j0 + 1, rows_b, sem_b)
                scatter(j0 + 1, rows_b)

            return carry

        lax.fori_loop(0, (nk + 1) // 2, pair, 0)
    plsc.subcore_barrier()
    pltpu.sync_copy(acc_sh.at[pl.ds(base, ROWS_PT)], out.at[c, pl.ds(base, ROWS_PT)])


_prop_call = pl.kernel(
    _prop_body,
    out_type=jax.ShapeDtypeStruct((2, NP, F), jnp.float32),
    compiler_params=_SC_PARAMS,
    mesh=_MESH,
    scratch_types=[
        pltpu.VMEM((PH, CH), jnp.int32),
        pltpu.VMEM((PH, CH), jnp.int32),
        pltpu.VMEM((CH, F), jnp.float32),
        pltpu.VMEM((CH, F), jnp.float32),
        pltpu.VMEM_SHARED((NP, F), jnp.float32),
        pltpu.SemaphoreType.DMA,
        pltpu.SemaphoreType.DMA,
    ],
)


def _pred_body(t_hbm, es_hbm, ed_hbm, out0, out1, t_v, es_v, ed_v, s0_v, s1_v):
    c = lax.axis_index("c")
    s = lax.axis_index("s")
    w = s * NC + c
    pltpu.sync_copy(t_hbm, t_v)
    pltpu.sync_copy(es_hbm.at[w], es_v)
    pltpu.sync_copy(ed_hbm.at[w], ed_v)

    def chunk(j, _):
        e4 = es_v[j] * 4
        d4 = ed_v[j] * 4
        g0 = plsc.load_gather(t_v, [e4])
        g1 = plsc.load_gather(t_v, [e4 + 1])
        g2 = plsc.load_gather(t_v, [d4 + 2])
        g3 = plsc.load_gather(t_v, [d4 + 3])
        s0_v[pl.ds(j * 16, 16)] = g0 + g2
        s1_v[pl.ds(j * 16, 16)] = g1 + g3
        return _

    lax.fori_loop(0, DCH, chunk, 0)
    pltpu.sync_copy(s0_v, out0.at[pl.ds(w * DPTP, DPTP)])
    pltpu.sync_copy(s1_v, out1.at[pl.ds(w * DPTP, DPTP)])


_pred_call = pl.kernel(
    _pred_body,
    out_type=(
        jax.ShapeDtypeStruct((NW * DPTP,), jnp.float32),
        jax.ShapeDtypeStruct((NW * DPTP,), jnp.float32),
    ),
    compiler_params=_SC_PARAMS,
    mesh=_MESH,
    scratch_types=[
        pltpu.VMEM((NP * 4,), jnp.float32),
        pltpu.VMEM((DCH, 16), jnp.int32),
        pltpu.VMEM((DCH, 16), jnp.int32),
        pltpu.VMEM((DPTP,), jnp.float32),
        pltpu.VMEM((DPTP,), jnp.float32),
    ],
)


RB = 1024
GRID = NP // RB
HB = RB // 128


def _rs_col(hv):
    return lax.rsqrt(jnp.maximum(hv, 1.0))


def _tc_feat_body(x_ref, h_ref, o_ref):
    hv = h_ref[...]
    x = x_ref[...]
    o_ref[0] = x * _rs_col(hv[0, 0])
    o_ref[1] = x * _rs_col(hv[1, 0])


def _tc_feats(xp, hist):
    return pl.pallas_call(
        _tc_feat_body,
        grid=(GRID,),
        in_specs=[
            pl.BlockSpec((RB, F), lambda i: (i, 0)),
            pl.BlockSpec((2, 2, RB, 1), lambda i: (0, 0, i, 0)),
        ],
        out_specs=pl.BlockSpec((2, RB, F), lambda i: (0, i, 0)),
        out_shape=jax.ShapeDtypeStruct((2, NP, F), jnp.float32),
    )(xp, hist)


def _tc_mid_body(a_ref, h_ref, w0_ref, w1_ref, b_ref, o_ref):
    hv = h_ref[...]
    y0 = a_ref[0] * _rs_col(hv[0, 1])
    y1 = a_ref[1] * _rs_col(hv[1, 1])
    h = jnp.dot(y0, w0_ref[...], preferred_element_type=jnp.float32)
    h += jnp.dot(y1, w1_ref[...], preferred_element_type=jnp.float32)
    h = jnp.maximum(h + b_ref[...], 0.0)
    o_ref[0] = h * _rs_col(hv[0, 0])
    o_ref[1] = h * _rs_col(hv[1, 0])


def _tc_mid(acc, hist, W1_0, W1_1, b1r):
    return pl.pallas_call(
        _tc_mid_body,
        grid=(GRID,),
        in_specs=[
            pl.BlockSpec((2, RB, F), lambda i: (0, i, 0)),
            pl.BlockSpec((2, 2, RB, 1), lambda i: (0, 0, i, 0)),
            pl.BlockSpec((F, F), lambda i: (0, 0)),
            pl.BlockSpec((F, F), lambda i: (0, 0)),
            pl.BlockSpec((1, F), lambda i: (0, 0)),
        ],
        out_specs=pl.BlockSpec((2, RB, F), lambda i: (0, i, 0)),
        out_shape=jax.ShapeDtypeStruct((2, NP, F), jnp.float32),
    )(acc, hist, W1_0, W1_1, b1r)


def _tc_final_body(a_ref, h_ref, w0_ref, w1_ref, b_ref, wp_ref, bp_ref, o_ref):
    hv = h_ref[...]
    y0 = a_ref[0] * _rs_col(hv[0, 1])
    y1 = a_ref[1] * _rs_col(hv[1, 1])
    h2 = jnp.dot(y0, w0_ref[...], preferred_element_type=jnp.float32)
    h2 += jnp.dot(y1, w1_ref[...], preferred_element_type=jnp.float32)
    h2 += b_ref[...]
    o_ref[...] = jnp.dot(h2, wp_ref[...], preferred_element_type=jnp.float32) + bp_ref[...]


def _tc_final(acc, hist, W2_0, W2_1, b2r, Wp4, bp4):
    return pl.pallas_call(
        _tc_final_body,
        grid=(GRID,),
        in_specs=[
            pl.BlockSpec((2, RB, F), lambda i: (0, i, 0)),
            pl.BlockSpec((2, 2, RB, 1), lambda i: (0, 0, i, 0)),
            pl.BlockSpec((F, F), lambda i: (0, 0)),
            pl.BlockSpec((F, F), lambda i: (0, 0)),
            pl.BlockSpec((1, F), lambda i: (0, 0)),
            pl.BlockSpec((F, 4), lambda i: (0, 0)),
            pl.BlockSpec((1, 4), lambda i: (0, 0)),
        ],
        out_specs=pl.BlockSpec((RB, 4), lambda i: (i, 0)),
        out_shape=jax.ShapeDtypeStruct((NP, 4), jnp.float32),
    )(acc, hist, W2_0, W2_1, b2r, Wp4, bp4)


def kernel(x, edge_index_rel0, edge_index_rel1, dec_edge_index,
           W1_0, b1_0, W1_1, b1_1, W2_0, b2_0, W2_1, b2_1, Wp, bp):
    xp = jnp.pad(x, ((0, NP - N), (0, 0)))

    def pack_rel(ei):
        return _pad_tile_idx(ei[0], DUMMY), _pad_tile_idx(ei[1], DUMMY)

    s0l, d0l = pack_rel(edge_index_rel0)
    s1l, d1l = pack_rel(edge_index_rel1)
    srcl = jnp.stack([s0l, s1l])
    dstl = jnp.stack([d0l, d1l])

    es = dec_edge_index[0].astype(jnp.int32).reshape(NW, DPT)
    ed = dec_edge_index[1].astype(jnp.int32).reshape(NW, DPT)
    es = jnp.pad(es, ((0, 0), (0, DPTP - DPT))).reshape(NW, DCH, 16)
    ed = jnp.pad(ed, ((0, 0), (0, DPTP - DPT))).reshape(NW, DCH, 16)

    zflat = jnp.zeros((HR, 128), jnp.float32)
    z128_hbm = jnp.zeros((ROWS_PT, F), jnp.float32)

    b1r = (b1_0 + b1_1).reshape(1, F)
    b2r = (b2_0 + b2_1).reshape(1, F)
    Wp4 = jnp.concatenate([Wp[:F], Wp[F:]], axis=1)
    bp4 = jnp.concatenate([bp, jnp.zeros((2,), jnp.float32)]).reshape(1, 4)

    hist = _hist_call(srcl, dstl, zflat)

    hist = hist.reshape(2, 2, NP, 1)

    feats = _tc_feats(xp, hist)
    acc1 = _prop_call(feats, srcl, dstl, z128_hbm)
    feats2 = _tc_mid(acc1, hist, W1_0, W1_1, b1r)

    acc2 = _prop_call(feats2, srcl, dstl, z128_hbm)
    t = _tc_final(acc2, hist, W2_0, W2_1, b2r, Wp4, bp4)

    o0, o1 = _pred_call(t.reshape(NP * 4), es, ed)
    o0 = o0.reshape(NW, DPTP)[:, :DPT].reshape(DEC)
    o1 = o1.reshape(NW, DPTP)[:, :DPT].reshape(DEC)
    return jnp.stack([o0, o1], axis=1)
